# baseline (speedup 1.0000x reference)
import functools

import jax
import jax.numpy as jnp
from jax.experimental import pallas as pl
from jax.experimental.pallas import tpu as pltpu


def _fused_body(h_ref, w_ref, b_ref, lt_ref, ids_ref, out_ref, flag_ref, *, bm, k):
    acc = jax.lax.dot_general(
        h_ref[...].astype(jnp.bfloat16), w_ref[...].astype(jnp.bfloat16),
        dimension_numbers=(((1,), (0,)), ((), ())),
        preferred_element_type=jnp.float32,
    )
    logits = acc + b_ref[...]
    ids = ids_ref[0, 0, :]
    iota = jax.lax.broadcasted_iota(jnp.int32, (bm, k), 1)
    inv = 1.0 / jnp.exp(lt_ref[...])
    inv_t = jnp.sum(jnp.where(ids[:, None] == iota, inv, 0.0), axis=1,
                    keepdims=True)
    scaled = logits * inv_t
    out_ref[...] = scaled
    fin = jnp.min(jnp.where(jnp.isfinite(scaled), jnp.float32(1), jnp.float32(0)))
    flag_ref[pl.program_id(0)] = fin


def _plain_body(h_ref, w_ref, b_ref, out_ref):
    acc = jax.lax.dot_general(
        h_ref[...], w_ref[...],
        dimension_numbers=(((1,), (0,)), ((), ())),
        preferred_element_type=jnp.float32,
    )
    out_ref[...] = acc + b_ref[...]


def _plain_logits(h, W, b2, *, bm, interpret=False):
    n, d = h.shape
    v = W.shape[1]
    nb = n // bm
    return pl.pallas_call(
        _plain_body,
        grid=(nb,),
        in_specs=[
            pl.BlockSpec((bm, d), lambda i: (i, 0)),
            pl.BlockSpec((d, v), lambda i: (0, 0)),
            pl.BlockSpec((1, v), lambda i: (0, 0)),
        ],
        out_specs=pl.BlockSpec((bm, v), lambda i: (i, 0)),
        out_shape=jax.ShapeDtypeStruct((n, v), jnp.float32),
        interpret=interpret,
    )(h, W, b2)


def kernel(h, topic_ids, W, b, log_T, interpret=False):
    n, d = h.shape
    v = W.shape[1]
    k = log_T.shape[0]
    bm = 256
    nb = n // bm
    ids3 = topic_ids.astype(jnp.int32).reshape(nb, 1, bm)
    b2 = b.reshape(1, v)
    lt2 = log_T.reshape(1, k)
    scaled, flags = pl.pallas_call(
        functools.partial(_fused_body, bm=bm, k=k),
        grid=(nb,),
        in_specs=[
            pl.BlockSpec((bm, d), lambda i: (i, 0)),
            pl.BlockSpec((d, v), lambda i: (0, 0)),
            pl.BlockSpec((1, v), lambda i: (0, 0)),
            pl.BlockSpec((1, k), lambda i: (0, 0)),
            pl.BlockSpec((1, 1, bm), lambda i: (i, 0, 0)),
        ],
        out_specs=[
            pl.BlockSpec((bm, v), lambda i: (i, 0)),
            pl.BlockSpec(memory_space=pltpu.SMEM),
        ],
        out_shape=[
            jax.ShapeDtypeStruct((n, v), jnp.float32),
            jax.ShapeDtypeStruct((nb,), jnp.float32),
        ],
        interpret=interpret,
    )(h, W, b2, lt2, ids3)
    all_finite = jnp.min(flags) > 0.0
    return jax.lax.cond(
        all_finite,
        lambda: scaled,
        lambda: _plain_logits(h, W, b2, bm=bm, interpret=interpret),
    )


# speedup vs baseline: 1.2258x; 1.2258x over previous
import functools

import jax
import jax.numpy as jnp
from jax.experimental import pallas as pl
from jax.experimental.pallas import tpu as pltpu


def _fused_body(h_ref, w_ref, b_ref, lt_ref, ids_ref, out_ref, flag_ref, *, bm, k):
    acc = jax.lax.dot_general(
        h_ref[...].astype(jnp.bfloat16), w_ref[...].astype(jnp.bfloat16),
        dimension_numbers=(((1,), (0,)), ((), ())),
        preferred_element_type=jnp.float32,
    )
    logits = acc + b_ref[...]
    ids = ids_ref[0, 0, :]
    iota = jax.lax.broadcasted_iota(jnp.int32, (bm, k), 1)
    inv = 1.0 / jnp.exp(lt_ref[...])
    inv_t = jnp.sum(jnp.where(ids[:, None] == iota, inv, 0.0), axis=1,
                    keepdims=True)
    scaled = logits * inv_t
    out_ref[...] = scaled
    fin = jnp.min(jnp.where(jnp.isfinite(scaled), jnp.float32(1), jnp.float32(0)))
    flag_ref[pl.program_id(0)] = fin


def _plain_body(h_ref, w_ref, b_ref, out_ref):
    acc = jax.lax.dot_general(
        h_ref[...], w_ref[...],
        dimension_numbers=(((1,), (0,)), ((), ())),
        preferred_element_type=jnp.float32,
    )
    out_ref[...] = acc + b_ref[...]


def _plain_logits(h, W, b2, *, bm, interpret=False):
    n, d = h.shape
    v = W.shape[1]
    nb = n // bm
    return pl.pallas_call(
        _plain_body,
        grid=(nb,),
        in_specs=[
            pl.BlockSpec((bm, d), lambda i: (i, 0)),
            pl.BlockSpec((d, v), lambda i: (0, 0)),
            pl.BlockSpec((1, v), lambda i: (0, 0)),
        ],
        out_specs=pl.BlockSpec((bm, v), lambda i: (i, 0)),
        out_shape=jax.ShapeDtypeStruct((n, v), jnp.float32),
        interpret=interpret,
    )(h, W, b2)


def kernel(h, topic_ids, W, b, log_T, interpret=False):
    n, d = h.shape
    v = W.shape[1]
    k = log_T.shape[0]
    bm = 1024
    nb = n // bm
    ids3 = topic_ids.astype(jnp.int32).reshape(nb, 1, bm)
    b2 = b.reshape(1, v)
    lt2 = log_T.reshape(1, k)
    scaled, flags = pl.pallas_call(
        functools.partial(_fused_body, bm=bm, k=k),
        grid=(nb,),
        in_specs=[
            pl.BlockSpec((bm, d), lambda i: (i, 0)),
            pl.BlockSpec((d, v), lambda i: (0, 0)),
            pl.BlockSpec((1, v), lambda i: (0, 0)),
            pl.BlockSpec((1, k), lambda i: (0, 0)),
            pl.BlockSpec((1, 1, bm), lambda i: (i, 0, 0)),
        ],
        out_specs=[
            pl.BlockSpec((bm, v), lambda i: (i, 0)),
            pl.BlockSpec(memory_space=pltpu.SMEM),
        ],
        out_shape=[
            jax.ShapeDtypeStruct((n, v), jnp.float32),
            jax.ShapeDtypeStruct((nb,), jnp.float32),
        ],
        interpret=interpret,
    )(h, W, b2, lt2, ids3)
    all_finite = jnp.min(flags) > 0.0
    return jax.lax.cond(
        all_finite,
        lambda: scaled,
        lambda: _plain_logits(h, W, b2, bm=bm, interpret=interpret),
    )
